# baseline (speedup 1.0000x reference)
import functools

import jax
import jax.numpy as jnp
from jax import lax
from jax.experimental import pallas as pl
from jax.experimental.pallas import tpu as pltpu
from jax.experimental.pallas import tpu_sc as plsc

_N = 256
_NH = 32
_NN = _N * _N
_ROWS = 961
_STRIDE = 968
_NW = 32
_CHUNK = _NN // _NW
_L = 16


def _sc_gather(table_t_flat, idx_flat):
    mesh = plsc.VectorSubcoreMesh(core_axis_name="c", subcore_axis_name="s")

    @functools.partial(
        pl.kernel,
        mesh=mesh,
        compiler_params=pltpu.CompilerParams(needs_layout_passes=False),
        out_type=jax.ShapeDtypeStruct((_NH, _NN // 1024, 8, 128), jnp.float32),
        scratch_types=[
            pltpu.VMEM((_STRIDE * _NH,), jnp.float32),
            pltpu.VMEM((8, _N), jnp.int32),
            pltpu.VMEM((_NH, 2, 8, 128), jnp.float32),
            pltpu.SemaphoreType.DMA,
            pltpu.SemaphoreType.DMA,
            pltpu.SemaphoreType.DMA,
            pltpu.SemaphoreType.DMA,
        ],
    )
    def body(
        table_hbm, idx_hbm, out_hbm, tab_v, idx_v, out_v, ts0, ts1, isem, osem
    ):
        wid = lax.axis_index("s") * 2 + lax.axis_index("c")
        half_tab = _STRIDE * _NH // 2

        cp_t0 = pltpu.async_copy(
            table_hbm.at[pl.ds(0, half_tab)], tab_v.at[pl.ds(0, half_tab)], ts0
        )
        cp_i = pltpu.async_copy(idx_hbm.at[pl.ds(wid * 8, 8)], idx_v, isem)
        cp_t1 = pltpu.async_copy(
            table_hbm.at[pl.ds(half_tab, half_tab)],
            tab_v.at[pl.ds(half_tab, half_tab)],
            ts1,
        )
        cp_i.wait()
        cp_t0.wait()

        copies = []
        for hg in range(2):
            if hg == 1:
                cp_t1.wait()
            for jb in range(2):

                @plsc.parallel_loop(0, _CHUNK // _L // 2, unroll=2)
                def step(t, _jb=jb, _hg=hg):
                    ii = t >> 3
                    ji = (t & 7) * _L
                    ivec = idx_v[ii, pl.ds(_jb * 128 + ji, _L)]
                    for h in range(_hg * 16, _hg * 16 + 16):
                        out_v[h, _jb, ii, pl.ds(ji, _L)] = plsc.load_gather(
                            tab_v.at[pl.ds(h * _STRIDE, _ROWS)], [ivec]
                        )

                copies.append(
                    pltpu.async_copy(
                        out_v.at[pl.ds(hg * 16, 16), pl.ds(jb, 1)],
                        out_hbm.at[pl.ds(hg * 16, 16), pl.ds(wid * 2 + jb, 1)],
                        osem,
                    )
                )
        for cp in copies:
            cp.wait()

    return body(table_t_flat, idx_flat)


def kernel(table, rel_index):
    table_t = jnp.pad(table.T, ((0, 0), (0, _STRIDE - _ROWS)))
    out = _sc_gather(table_t.reshape(-1), rel_index)
    out = out.reshape(_NH, _N // 8, _N // 128, 8, 128)
    out = out.transpose(0, 1, 3, 2, 4)
    return out.reshape(1, _NH, _N, _N)


# speedup vs baseline: 1.0534x; 1.0534x over previous
import functools

import jax
import jax.numpy as jnp
from jax import lax
from jax.experimental import pallas as pl
from jax.experimental.pallas import tpu as pltpu
from jax.experimental.pallas import tpu_sc as plsc

_N = 256
_NH = 32
_NN = _N * _N
_ROWS = 961
_STRIDE = 968
_NW = 32
_CHUNK = _NN // _NW
_L = 16


def _sc_gather(table_t_flat, idx_flat):
    mesh = plsc.VectorSubcoreMesh(core_axis_name="c", subcore_axis_name="s")

    @functools.partial(
        pl.kernel,
        mesh=mesh,
        compiler_params=pltpu.CompilerParams(needs_layout_passes=False),
        out_type=jax.ShapeDtypeStruct((_NH, _NN // 1024, 8, 128), jnp.float32),
        scratch_types=[
            pltpu.VMEM((_STRIDE * _NH,), jnp.float32),
            pltpu.VMEM((8, _N), jnp.int32),
            pltpu.VMEM((_NH, 2, 8, 128), jnp.float32),
            pltpu.SemaphoreType.DMA,
            pltpu.SemaphoreType.DMA,
            pltpu.SemaphoreType.DMA,
            pltpu.SemaphoreType.DMA,
        ],
    )
    def body(
        table_hbm, idx_hbm, out_hbm, tab_v, idx_v, out_v, ts0, ts1, isem, osem
    ):
        wid = lax.axis_index("s") * 2 + lax.axis_index("c")
        half_tab = _STRIDE * _NH // 2

        cp_t0 = pltpu.async_copy(
            table_hbm.at[pl.ds(0, half_tab)], tab_v.at[pl.ds(0, half_tab)], ts0
        )
        cp_i = pltpu.async_copy(idx_hbm.at[pl.ds(wid * 8, 8)], idx_v, isem)
        cp_t1 = pltpu.async_copy(
            table_hbm.at[pl.ds(half_tab, half_tab)],
            tab_v.at[pl.ds(half_tab, half_tab)],
            ts1,
        )
        cp_i.wait()
        cp_t0.wait()

        def quarter(q, carry):
            hb = (q >> 1) << 4
            jb = q & 1

            @pl.when(q == 2)
            def _wait_t1():
                cp_t1.wait()

            @plsc.parallel_loop(0, _CHUNK // _L // 2, unroll=2)
            def step(t):
                ii = t >> 3
                ji = (t & 7) * _L
                ivec = idx_v[ii, pl.ds(jb * 128 + ji, _L)]
                for k in range(16):
                    out_v[hb + k, jb, ii, pl.ds(ji, _L)] = plsc.load_gather(
                        tab_v.at[pl.ds((hb + k) * _STRIDE, _ROWS)], [ivec]
                    )

            pltpu.async_copy(
                out_v.at[pl.ds(hb, 16), pl.ds(jb, 1)],
                out_hbm.at[pl.ds(hb, 16), pl.ds(wid * 2 + jb, 1)],
                osem,
            )
            return carry

        lax.fori_loop(0, 4, quarter, 0)
        for _ in range(4):
            pltpu.make_async_copy(
                out_hbm.at[pl.ds(0, 16), pl.ds(0, 1)],
                out_v.at[pl.ds(0, 16), pl.ds(0, 1)],
                osem,
            ).wait()

    return body(table_t_flat, idx_flat)


def kernel(table, rel_index):
    table_t = jnp.pad(table.T, ((0, 0), (0, _STRIDE - _ROWS)))
    out = _sc_gather(table_t.reshape(-1), rel_index)
    out = out.reshape(_NH, _N // 8, _N // 128, 8, 128)
    out = out.transpose(0, 1, 3, 2, 4)
    return out.reshape(1, _NH, _N, _N)
